# baseline (speedup 1.0000x reference)
import functools

import jax
import jax.numpy as jnp
from jax.experimental import pallas as pl
from jax.experimental.pallas import tpu as pltpu

_EPS = 1e-5
_C = 32
_PB = 16


def _block_kernel(nb, h, W, C,
                  x_ref, cx_ref, ws_ref,
                  g1_ref, b1_ref, g2_ref, b2_ref, bc_ref,
                  res_ref, cpm_ref, pres_ref, pcpm_ref, wband_ref, mavg_ref):
    wc = x_ref.shape[-1]
    f32, bf16 = jnp.float32, jnp.bfloat16

    @pl.when(pl.program_id(1) == 0)
    def _fold_weights():
        ri = jax.lax.broadcasted_iota(jnp.int32, (wc, wc), 0)
        ci = jax.lax.broadcasted_iota(jnp.int32, (wc, wc), 1)
        mavg_ref[...] = jnp.where((ri % C) == (ci % C),
                                  1.0 / (h * W), 0.0).astype(f32)
        zrow = jnp.zeros((nb, 1, wc), bf16)
        pres_ref[:, _PB - 1:_PB, :] = zrow
        pres_ref[:, _PB + h:_PB + h + 1, :] = zrow
        pcpm_ref[:, _PB - 1:_PB, :] = zrow
        pcpm_ref[:, _PB + h:_PB + h + 1, :] = zrow
        E = (jax.lax.broadcasted_iota(jnp.int32, (wc, C), 0) % C
             == jax.lax.broadcasted_iota(jnp.int32, (wc, C), 1)).astype(bf16)
        Et = (jax.lax.broadcasted_iota(jnp.int32, (C, wc), 0)
              == jax.lax.broadcasted_iota(jnp.int32, (C, wc), 1) % C
              ).astype(bf16)
        diag = (ri // C) == (ci // C)
        zr = jnp.zeros((C, wc), bf16)
        for c in range(3):
            for kh in range(3):
                ts = []
                for kw in range(3):
                    wk = ws_ref[pl.ds((kh * 3 + kw) * wc, C),
                                pl.ds(c * C, C)]
                    q = jnp.dot(wk, Et, preferred_element_type=f32)
                    p = jnp.dot(E, q.astype(bf16), preferred_element_type=f32)
                    ts.append(jnp.where(diag, p.astype(bf16), 0))
                band = (ts[1]
                        + jnp.concatenate([ts[0][C:, :], zr], axis=0)
                        + jnp.concatenate([zr, ts[2][:wc - C, :]], axis=0))
                wband_ref[c * 3 + kh] = band

    def conv(pad_ref, base):
        acc = jnp.dot(pad_ref[:, _PB - 1:_PB - 1 + h, :].reshape(nb * h, wc),
                      wband_ref[base], preferred_element_type=f32)
        for kh in (1, 2):
            acc += jnp.dot(
                pad_ref[:, _PB - 1 + kh:_PB - 1 + kh + h, :].reshape(nb * h, wc),
                wband_ref[base + kh], preferred_element_type=f32)
        return acc

    def inorm(y, g, b):
        y3 = y.reshape(nb, h, wc)
        s1 = jnp.sum(y3, axis=1)
        s2 = jnp.sum(y3 * y3, axis=1)
        st = jnp.dot(jnp.concatenate([s1, s2], axis=0), mavg_ref[...],
                     preferred_element_type=f32)
        mean, ms = st[:nb], st[nb:]
        scale = g * jax.lax.rsqrt(ms - mean * mean + _EPS)
        shift = b - mean * scale
        return y3 * scale[:, None, :] + shift[:, None, :]

    pres_ref[:, _PB:_PB + h, :] = x_ref[...].astype(bf16)
    pcpm_ref[:, _PB:_PB + h, :] = cx_ref[...].astype(bf16)
    y1 = jnp.maximum(inorm(conv(pres_ref, 0), g1_ref[...], b1_ref[...]), 0.0)
    pres_ref[:, _PB:_PB + h, :] = y1.astype(bf16)
    c2 = conv(pres_ref, 3)
    yc = conv(pcpm_ref, 6).reshape(nb, h, wc)
    cpm_ref[...] = jnp.maximum(yc + bc_ref[...][None, :, :], 0.0)
    y2 = inorm(c2, g2_ref[...], b2_ref[...])
    res_ref[...] = jnp.maximum(x_ref[...] + y2, 0.0)


def kernel(x2d, cx2d, w1b, w2b, wcb, mavg, g1t, b1t, g2t, b2t, bct):
    N, H, WC = x2d.shape
    C = _C
    W = WC // C
    f32 = jnp.float32
    nb = next(b for b in (16, 8, 4, 2, 1) if N % b == 0)
    steps = N // nb
    ncore = 2 if steps % 2 == 0 else 1
    inner = steps // ncore
    ws = jnp.concatenate([w1b[:, :C], w2b[:, :C], wcb[:, :C]], axis=1)

    io_spec = pl.BlockSpec((nb, H, WC), lambda o, i, _g=inner: (o * _g + i, 0, 0))

    def const_spec(a):
        nd = a.ndim
        idx = lambda o, i, _nd=nd: (0,) * _nd
        try:
            return pl.BlockSpec(a.shape, idx, pipeline_mode=pl.Buffered(1))
        except Exception:
            return pl.BlockSpec(a.shape, idx)

    res, cpm = pl.pallas_call(
        functools.partial(_block_kernel, nb, H, W, C),
        out_shape=(jax.ShapeDtypeStruct((N, H, WC), f32),
                   jax.ShapeDtypeStruct((N, H, WC), f32)),
        grid=(ncore, inner),
        in_specs=[io_spec, io_spec]
                 + [const_spec(a) for a in (ws,
                                            g1t, b1t, g2t, b2t, bct)],
        out_specs=(io_spec, io_spec),
        scratch_shapes=[pltpu.VMEM((nb, H + 2 * _PB, WC), jnp.bfloat16),
                        pltpu.VMEM((nb, H + 2 * _PB, WC), jnp.bfloat16),
                        pltpu.VMEM((9, WC, WC), jnp.bfloat16),
                        pltpu.VMEM((WC, WC), f32)],
        compiler_params=pltpu.CompilerParams(
            dimension_semantics=("parallel", "arbitrary"),
            vmem_limit_bytes=64 * 1024 * 1024),
    )(x2d, cx2d, ws, g1t, b1t, g2t, b2t, bct)
    return res, cpm


# speedup vs baseline: 1.2685x; 1.2685x over previous
import functools

import jax
import jax.numpy as jnp
from jax.experimental import pallas as pl
from jax.experimental.pallas import tpu as pltpu

_EPS = 1e-5
_C = 32
_PB = 16


def _block_kernel(nb, h, W, C,
                  x_ref, cx_ref, w1b_ref, w2b_ref, wcb_ref,
                  g1_ref, b1_ref, g2_ref, b2_ref, bc_ref,
                  res_ref, cpm_ref, pres_ref, pcpm_ref, wband_ref, mavg_ref):
    wc = x_ref.shape[-1]
    f32, bf16 = jnp.float32, jnp.bfloat16

    @pl.when(pl.program_id(1) == 0)
    def _fold_weights():
        ri = jax.lax.broadcasted_iota(jnp.int32, (wc, wc), 0)
        ci = jax.lax.broadcasted_iota(jnp.int32, (wc, wc), 1)
        mavg_ref[...] = jnp.where((ri % C) == (ci % C),
                                  1.0 / (h * W), 0.0).astype(f32)
        zrow = jnp.zeros((nb, 1, wc), bf16)
        pres_ref[:, _PB - 1:_PB, :] = zrow
        pres_ref[:, _PB + h:_PB + h + 1, :] = zrow
        pcpm_ref[:, _PB - 1:_PB, :] = zrow
        pcpm_ref[:, _PB + h:_PB + h + 1, :] = zrow
        zc = jnp.zeros((wc, C), bf16)
        for c, wb_ref in enumerate((w1b_ref, w2b_ref, wcb_ref)):
            for kh in range(3):
                t0 = wb_ref[pl.ds((kh * 3 + 0) * wc, wc), :]
                t1 = wb_ref[pl.ds((kh * 3 + 1) * wc, wc), :]
                t2 = wb_ref[pl.ds((kh * 3 + 2) * wc, wc), :]
                band = (t1
                        + jnp.concatenate([zc, t0[:, :wc - C]], axis=1)
                        + jnp.concatenate([t2[:, C:], zc], axis=1))
                wband_ref[c * 3 + kh] = band

    def conv(pad_ref, base):
        acc = jnp.dot(pad_ref[:, _PB - 1:_PB - 1 + h, :].reshape(nb * h, wc),
                      wband_ref[base], preferred_element_type=f32)
        for kh in (1, 2):
            acc += jnp.dot(
                pad_ref[:, _PB - 1 + kh:_PB - 1 + kh + h, :].reshape(nb * h, wc),
                wband_ref[base + kh], preferred_element_type=f32)
        return acc

    def inorm(y, g, b):
        y3 = y.reshape(nb, h, wc)
        s1 = jnp.sum(y3, axis=1)
        s2 = jnp.sum(y3 * y3, axis=1)
        st = jnp.dot(jnp.concatenate([s1, s2], axis=0), mavg_ref[...],
                     preferred_element_type=f32)
        mean, ms = st[:nb], st[nb:]
        scale = g * jax.lax.rsqrt(ms - mean * mean + _EPS)
        shift = b - mean * scale
        return y3 * scale[:, None, :] + shift[:, None, :]

    pres_ref[:, _PB:_PB + h, :] = x_ref[...].astype(bf16)
    pcpm_ref[:, _PB:_PB + h, :] = cx_ref[...].astype(bf16)
    y1 = jnp.maximum(inorm(conv(pres_ref, 0), g1_ref[...], b1_ref[...]), 0.0)
    pres_ref[:, _PB:_PB + h, :] = y1.astype(bf16)
    c2 = conv(pres_ref, 3)
    yc = conv(pcpm_ref, 6).reshape(nb, h, wc)
    cpm_ref[...] = jnp.maximum(yc + bc_ref[...][None, :, :], 0.0)
    y2 = inorm(c2, g2_ref[...], b2_ref[...])
    res_ref[...] = jnp.maximum(x_ref[...] + y2, 0.0)


def kernel(x2d, cx2d, w1b, w2b, wcb, mavg, g1t, b1t, g2t, b2t, bct):
    N, H, WC = x2d.shape
    C = _C
    W = WC // C
    f32 = jnp.float32
    nb = next(b for b in (16, 8, 4, 2, 1) if N % b == 0)
    steps = N // nb
    ncore = 2 if steps % 2 == 0 else 1
    inner = steps // ncore

    io_spec = pl.BlockSpec((nb, H, WC), lambda o, i, _g=inner: (o * _g + i, 0, 0))

    def const_spec(a):
        nd = a.ndim
        idx = lambda o, i, _nd=nd: (0,) * _nd
        try:
            return pl.BlockSpec(a.shape, idx, pipeline_mode=pl.Buffered(1))
        except Exception:
            return pl.BlockSpec(a.shape, idx)

    res, cpm = pl.pallas_call(
        functools.partial(_block_kernel, nb, H, W, C),
        out_shape=(jax.ShapeDtypeStruct((N, H, WC), f32),
                   jax.ShapeDtypeStruct((N, H, WC), f32)),
        grid=(ncore, inner),
        in_specs=[io_spec, io_spec]
                 + [const_spec(a) for a in (w1b, w2b, wcb,
                                            g1t, b1t, g2t, b2t, bct)],
        out_specs=(io_spec, io_spec),
        scratch_shapes=[pltpu.VMEM((nb, H + 2 * _PB, WC), jnp.bfloat16),
                        pltpu.VMEM((nb, H + 2 * _PB, WC), jnp.bfloat16),
                        pltpu.VMEM((9, WC, WC), jnp.bfloat16),
                        pltpu.VMEM((WC, WC), f32)],
        compiler_params=pltpu.CompilerParams(
            dimension_semantics=("parallel", "arbitrary"),
            vmem_limit_bytes=64 * 1024 * 1024),
    )(x2d, cx2d, w1b, w2b, wcb, g1t, b1t, g2t, b2t, bct)
    return res, cpm
